# baseline (speedup 1.0000x reference)
import functools

import jax
import jax.numpy as jnp
from jax import lax
from jax.experimental import pallas as pl
from jax.experimental.pallas import tpu as pltpu
from jax.experimental.pallas import tpu_sc as plsc

N = 10000
E = 320000
D = 128
NC = 2
NS = 16
NW = NC * NS
EPW = E // NW
C = 80
IB = 10
RB = 400
RPS = 624


def _proj_body(x_ref, w_ref, asrc_ref, adst_ref, h_ref, sa_ref, sd_ref):
    x = x_ref[...]
    h = jnp.dot(x, w_ref[...], preferred_element_type=jnp.float32)
    h_ref[...] = h
    sa_ref[...] = jnp.dot(h, asrc_ref[...], preferred_element_type=jnp.float32)
    sd_ref[...] = jnp.dot(h, adst_ref[...], preferred_element_type=jnp.float32)


def _project(x, w, asrc, adst):
    return pl.pallas_call(
        _proj_body,
        grid=(N // RB,),
        in_specs=[
            pl.BlockSpec((RB, D), lambda i: (i, 0)),
            pl.BlockSpec((D, D), lambda i: (0, 0)),
            pl.BlockSpec((D, 1), lambda i: (0, 0)),
            pl.BlockSpec((D, 1), lambda i: (0, 0)),
        ],
        out_specs=[
            pl.BlockSpec((RB, D), lambda i: (i, 0)),
            pl.BlockSpec((RB, 1), lambda i: (i, 0)),
            pl.BlockSpec((RB, 1), lambda i: (i, 0)),
        ],
        out_shape=[
            jax.ShapeDtypeStruct((N, D), jnp.float32),
            jax.ShapeDtypeStruct((N, 1), jnp.float32),
            jax.ShapeDtypeStruct((N, 1), jnp.float32),
        ],
    )(x, w, asrc, adst)


def _sc_body(h_hbm, eidx_hbm, sa_hbm, sd_hbm, num_hbm, den_hbm,
             as_v, ad_v, sidi0_v, sidi1_v, ex_v, rows0_v, rows1_v,
             den_v, acc, sem0, sem1):
    cid = lax.axis_index("c")
    sid = lax.axis_index("s")
    wid = sid * NC + cid

    pltpu.sync_copy(sa_hbm, as_v)
    pltpu.sync_copy(sd_hbm, ad_v)

    @pl.loop(0, N // 16)
    def _(i):
        den_v[pl.ds(i * 16, 16)] = jnp.zeros((16,), jnp.float32)

    @pl.loop(0, C)
    def _(r):
        for cb in range(D // 16):
            rows0_v[r, pl.ds(cb * 16, 16)] = jnp.zeros((16,), jnp.float32)

    @pl.loop(0, RPS // C)
    def _(j):
        pltpu.sync_copy(rows0_v, acc.at[pl.ds(sid * RPS + j * C, C)])

    pltpu.sync_copy(rows0_v.at[pl.ds(0, RPS - (RPS // C) * C)],
                    acc.at[pl.ds(sid * RPS + (RPS // C) * C,
                                 RPS - (RPS // C) * C)])

    @pl.when(sid == NS - 1)
    def _():
        pltpu.sync_copy(rows0_v.at[pl.ds(0, N - NS * RPS)],
                        acc.at[pl.ds(NS * RPS, N - NS * RPS)])

    def _mbody(i, m):
        return jnp.maximum(m, as_v[pl.ds(i * 16, 16)])

    mvec = lax.fori_loop(0, N // 16, _mbody,
                         jnp.full((16,), -3e38, jnp.float32))
    amax = jnp.max(mvec)

    plsc.subcore_barrier()

    NCHUNK = EPW // C
    ibase = wid * NCHUNK

    def fetch(t, sidi, rows, sem):
        pltpu.sync_copy(eidx_hbm.at[ibase + t], sidi)
        pltpu.async_copy(h_hbm.at[sidi.at[0]], rows, sem)

    def process(sidi, rows, sem):
        for g in range(C // 16):
            sidx = sidi[0, pl.ds(g * 16, 16)]
            didx = sidi[1, pl.ds(g * 16, 16)]
            sv = plsc.load_gather(as_v, [sidx])
            dv = plsc.load_gather(ad_v, [didx])
            v = sv + dv
            e = jnp.maximum(v, 0.2 * v)
            w = amax + dv
            cmax = jnp.maximum(w, 0.2 * w)
            ex = jnp.exp(e - cmax)
            ex_v[pl.ds(g * 16, 16)] = ex
            plsc.addupdate_scatter(den_v, [didx], ex)

        pltpu.make_async_copy(h_hbm.at[sidi.at[0]], rows, sem).wait()

        @pl.loop(0, C // 8)
        def _(r8):
            r0 = r8 * 8
            for k in range(8):
                r = r0 + k
                exb = plsc.load_gather(ex_v, [jnp.full((16,), r, jnp.int32)])
                for cb in range(D // 16):
                    rows[r, pl.ds(cb * 16, 16)] = (
                        rows[r, pl.ds(cb * 16, 16)] * exb)

        pltpu.sync_copy(rows, acc.at[sidi.at[1]], add=True)

    fetch(0, sidi0_v, rows0_v, sem0)

    @pl.loop(0, (NCHUNK + 1) // 2)
    def _(p):
        t0 = 2 * p

        @pl.when(t0 + 1 <= NCHUNK - 1)
        def _():
            fetch(t0 + 1, sidi1_v, rows1_v, sem1)

        process(sidi0_v, rows0_v, sem0)

        @pl.when(t0 + 2 <= NCHUNK - 1)
        def _():
            fetch(t0 + 2, sidi0_v, rows0_v, sem0)

        @pl.when(t0 + 1 <= NCHUNK - 1)
        def _():
            process(sidi1_v, rows1_v, sem1)

    plsc.subcore_barrier()

    pltpu.sync_copy(acc.at[pl.ds(sid * RPS, RPS)],
                    num_hbm.at[cid, pl.ds(sid * RPS, RPS)])

    @pl.when(sid == NS - 1)
    def _():
        pltpu.sync_copy(acc.at[pl.ds(NS * RPS, N - NS * RPS)],
                        num_hbm.at[cid, pl.ds(NS * RPS, N - NS * RPS)])

    pltpu.sync_copy(den_v, den_hbm.at[cid, sid])


def _sc_edge(h, eidx, sa, sd):
    mesh = plsc.VectorSubcoreMesh(core_axis_name="c", subcore_axis_name="s")
    k = functools.partial(
        pl.kernel,
        out_type=[
            jax.ShapeDtypeStruct((NC, N, D), jnp.float32),
            jax.ShapeDtypeStruct((NC, NS, N), jnp.float32),
        ],
        mesh=mesh,
        compiler_params=pltpu.CompilerParams(needs_layout_passes=False,
                                             use_tc_tiling_on_sc=False),
        scratch_types=[
            pltpu.VMEM((N,), jnp.float32),
            pltpu.VMEM((N,), jnp.float32),
            pltpu.VMEM((2, C), jnp.int32),
            pltpu.VMEM((2, C), jnp.int32),
            pltpu.VMEM((C,), jnp.float32),
            pltpu.VMEM((C, D), jnp.float32),
            pltpu.VMEM((C, D), jnp.float32),
            pltpu.VMEM((N,), jnp.float32),
            pltpu.VMEM_SHARED((N, D), jnp.float32),
            pltpu.SemaphoreType.DMA,
            pltpu.SemaphoreType.DMA,
        ],
    )(_sc_body)
    return k(h, eidx, sa, sd)


def _combine_body(p_ref, dens_ref, bias_ref, out_ref):
    num = p_ref[0] + p_ref[1]
    den = jnp.sum(dens_ref[...], axis=(0, 1))
    out_ref[...] = num / (den[:, None] + 1e-16) + bias_ref[...]


def _combine(parts, dens, bias2d):
    return pl.pallas_call(
        _combine_body,
        out_shape=jax.ShapeDtypeStruct((N, D), jnp.float32),
    )(parts, dens, bias2d)


def kernel(x, edge_index, W, a_src, a_dst, bias):
    eidx = jnp.transpose(edge_index.reshape(2, E // C, C), (1, 0, 2))
    h, sa, sd = _project(x, W, a_src.reshape(D, 1), a_dst.reshape(D, 1))
    parts, dens = _sc_edge(h, eidx, sa.reshape(N), sd.reshape(N))
    return _combine(parts, dens, bias.reshape(1, D))


# speedup vs baseline: 1.1332x; 1.1332x over previous
import functools

import jax
import jax.numpy as jnp
from jax import lax
from jax.experimental import pallas as pl
from jax.experimental.pallas import tpu as pltpu
from jax.experimental.pallas import tpu_sc as plsc

N = 10000
E = 320000
D = 128
NC = 2
NS = 16
NW = NC * NS
EPW = E // NW
C = 80
IB = 10
RB = 400
RPS = 624


def _proj_body(x_ref, w_ref, asrc_ref, adst_ref, h_ref, sa_ref, sd_ref):
    x = x_ref[...]
    h = jnp.dot(x, w_ref[...], preferred_element_type=jnp.float32)
    h_ref[...] = h
    sa_ref[...] = jnp.dot(h, asrc_ref[...], preferred_element_type=jnp.float32)
    sd_ref[...] = jnp.dot(h, adst_ref[...], preferred_element_type=jnp.float32)


def _project(x, w, asrc, adst):
    return pl.pallas_call(
        _proj_body,
        grid=(N // RB,),
        in_specs=[
            pl.BlockSpec((RB, D), lambda i: (i, 0)),
            pl.BlockSpec((D, D), lambda i: (0, 0)),
            pl.BlockSpec((D, 1), lambda i: (0, 0)),
            pl.BlockSpec((D, 1), lambda i: (0, 0)),
        ],
        out_specs=[
            pl.BlockSpec((RB, D), lambda i: (i, 0)),
            pl.BlockSpec((RB, 1), lambda i: (i, 0)),
            pl.BlockSpec((RB, 1), lambda i: (i, 0)),
        ],
        out_shape=[
            jax.ShapeDtypeStruct((N, D), jnp.float32),
            jax.ShapeDtypeStruct((N, 1), jnp.float32),
            jax.ShapeDtypeStruct((N, 1), jnp.float32),
        ],
    )(x, w, asrc, adst)


def _sc_body(h_hbm, eidx_hbm, sa_hbm, sd_hbm, num_hbm, den_hbm,
             as_v, ad_v, sidi0_v, sidi1_v, ex_v, rows0_v, rows1_v,
             den_v, acc, sem0, sem1):
    cid = lax.axis_index("c")
    sid = lax.axis_index("s")
    wid = sid * NC + cid

    pltpu.sync_copy(sa_hbm, as_v)
    pltpu.sync_copy(sd_hbm, ad_v)

    @pl.loop(0, N // 16)
    def _(i):
        den_v[pl.ds(i * 16, 16)] = jnp.zeros((16,), jnp.float32)

    @pl.loop(0, C)
    def _(r):
        for cb in range(D // 16):
            rows0_v[r, pl.ds(cb * 16, 16)] = jnp.zeros((16,), jnp.float32)

    @pl.loop(0, RPS // C)
    def _(j):
        pltpu.sync_copy(rows0_v, acc.at[pl.ds(sid * RPS + j * C, C)])

    pltpu.sync_copy(rows0_v.at[pl.ds(0, RPS - (RPS // C) * C)],
                    acc.at[pl.ds(sid * RPS + (RPS // C) * C,
                                 RPS - (RPS // C) * C)])

    @pl.when(sid == NS - 1)
    def _():
        pltpu.sync_copy(rows0_v.at[pl.ds(0, N - NS * RPS)],
                        acc.at[pl.ds(NS * RPS, N - NS * RPS)])

    def _mbody(i, m):
        return jnp.maximum(m, as_v[pl.ds(i * 16, 16)])

    mvec = lax.fori_loop(0, N // 16, _mbody,
                         jnp.full((16,), -3e38, jnp.float32))
    amax = jnp.max(mvec)

    plsc.subcore_barrier()

    NCHUNK = EPW // C
    ibase = wid * NCHUNK

    def fetch(t, sidi, rows, sem):
        pltpu.sync_copy(eidx_hbm.at[ibase + t], sidi)
        pltpu.async_copy(h_hbm.at[sidi.at[0]], rows, sem)

    def process(sidi, rows, sem):
        for g in range(C // 16):
            sidx = sidi[0, pl.ds(g * 16, 16)]
            didx = sidi[1, pl.ds(g * 16, 16)]
            sv = plsc.load_gather(as_v, [sidx])
            dv = plsc.load_gather(ad_v, [didx])
            v = sv + dv
            e = jnp.maximum(v, 0.2 * v)
            w = amax + dv
            cmax = jnp.maximum(w, 0.2 * w)
            ex = jnp.exp(e - cmax)
            ex_v[pl.ds(g * 16, 16)] = ex
            plsc.addupdate_scatter(den_v, [didx], ex)

        pltpu.make_async_copy(h_hbm.at[sidi.at[0]], rows, sem).wait()

        @plsc.parallel_loop(0, C, step=1, unroll=8)
        def _(r):
            exb = plsc.load_gather(ex_v, [jnp.full((16,), r, jnp.int32)])
            for cb in range(D // 16):
                rows[r, pl.ds(cb * 16, 16)] = (
                    rows[r, pl.ds(cb * 16, 16)] * exb)

        pltpu.sync_copy(rows, acc.at[sidi.at[1]], add=True)

    fetch(0, sidi0_v, rows0_v, sem0)

    @pl.loop(0, (NCHUNK + 1) // 2)
    def _(p):
        t0 = 2 * p

        @pl.when(t0 + 1 <= NCHUNK - 1)
        def _():
            fetch(t0 + 1, sidi1_v, rows1_v, sem1)

        process(sidi0_v, rows0_v, sem0)

        @pl.when(t0 + 2 <= NCHUNK - 1)
        def _():
            fetch(t0 + 2, sidi0_v, rows0_v, sem0)

        @pl.when(t0 + 1 <= NCHUNK - 1)
        def _():
            process(sidi1_v, rows1_v, sem1)

    plsc.subcore_barrier()

    pltpu.sync_copy(acc.at[pl.ds(sid * RPS, RPS)],
                    num_hbm.at[cid, pl.ds(sid * RPS, RPS)])

    @pl.when(sid == NS - 1)
    def _():
        pltpu.sync_copy(acc.at[pl.ds(NS * RPS, N - NS * RPS)],
                        num_hbm.at[cid, pl.ds(NS * RPS, N - NS * RPS)])

    pltpu.sync_copy(den_v, den_hbm.at[cid, sid])


def _sc_edge(h, eidx, sa, sd):
    mesh = plsc.VectorSubcoreMesh(core_axis_name="c", subcore_axis_name="s")
    k = functools.partial(
        pl.kernel,
        out_type=[
            jax.ShapeDtypeStruct((NC, N, D), jnp.float32),
            jax.ShapeDtypeStruct((NC, NS, N), jnp.float32),
        ],
        mesh=mesh,
        compiler_params=pltpu.CompilerParams(needs_layout_passes=False,
                                             use_tc_tiling_on_sc=False),
        scratch_types=[
            pltpu.VMEM((N,), jnp.float32),
            pltpu.VMEM((N,), jnp.float32),
            pltpu.VMEM((2, C), jnp.int32),
            pltpu.VMEM((2, C), jnp.int32),
            pltpu.VMEM((C,), jnp.float32),
            pltpu.VMEM((C, D), jnp.float32),
            pltpu.VMEM((C, D), jnp.float32),
            pltpu.VMEM((N,), jnp.float32),
            pltpu.VMEM_SHARED((N, D), jnp.float32),
            pltpu.SemaphoreType.DMA,
            pltpu.SemaphoreType.DMA,
        ],
    )(_sc_body)
    return k(h, eidx, sa, sd)


def _combine_body(p_ref, dens_ref, bias_ref, out_ref):
    num = p_ref[0] + p_ref[1]
    den = jnp.sum(dens_ref[...], axis=(0, 1))
    out_ref[...] = num / (den[:, None] + 1e-16) + bias_ref[...]


def _combine(parts, dens, bias2d):
    return pl.pallas_call(
        _combine_body,
        out_shape=jax.ShapeDtypeStruct((N, D), jnp.float32),
    )(parts, dens, bias2d)


def kernel(x, edge_index, W, a_src, a_dst, bias):
    eidx = jnp.transpose(edge_index.reshape(2, E // C, C), (1, 0, 2))
    h, sa, sd = _project(x, W, a_src.reshape(D, 1), a_dst.reshape(D, 1))
    parts, dens = _sc_edge(h, eidx, sa.reshape(N), sd.reshape(N))
    return _combine(parts, dens, bias.reshape(1, D))


# speedup vs baseline: 1.1553x; 1.0195x over previous
import functools

import jax
import jax.numpy as jnp
from jax import lax
from jax.experimental import pallas as pl
from jax.experimental.pallas import tpu as pltpu
from jax.experimental.pallas import tpu_sc as plsc

N = 10000
E = 320000
D = 128
NC = 2
NS = 16
NW = NC * NS
EPW = E // NW
C = 80
IB = 10
RB = 400
RPS = 624


def _proj_body(x_ref, w_ref, asrc_ref, adst_ref, h_ref, sa_ref, sd_ref):
    x = x_ref[...]
    h = jnp.dot(x, w_ref[...], preferred_element_type=jnp.float32)
    h_ref[...] = h
    sa_ref[...] = jnp.dot(h, asrc_ref[...], preferred_element_type=jnp.float32)
    sd_ref[...] = jnp.dot(h, adst_ref[...], preferred_element_type=jnp.float32)


def _project(x, w, asrc, adst):
    return pl.pallas_call(
        _proj_body,
        grid=(N // RB,),
        in_specs=[
            pl.BlockSpec((RB, D), lambda i: (i, 0)),
            pl.BlockSpec((D, D), lambda i: (0, 0)),
            pl.BlockSpec((D, 1), lambda i: (0, 0)),
            pl.BlockSpec((D, 1), lambda i: (0, 0)),
        ],
        out_specs=[
            pl.BlockSpec((RB, D), lambda i: (i, 0)),
            pl.BlockSpec((RB, 1), lambda i: (i, 0)),
            pl.BlockSpec((RB, 1), lambda i: (i, 0)),
        ],
        out_shape=[
            jax.ShapeDtypeStruct((N, D), jnp.float32),
            jax.ShapeDtypeStruct((N, 1), jnp.float32),
            jax.ShapeDtypeStruct((N, 1), jnp.float32),
        ],
    )(x, w, asrc, adst)


def _sc_body(h_hbm, eidx_hbm, sa_hbm, sd_hbm, num_hbm, den_hbm,
             as_v, ad_v, sidi0_v, sidi1_v, ex_v, rows0_v, rows1_v,
             den_v, acc, sem0, sem1):
    cid = lax.axis_index("c")
    sid = lax.axis_index("s")
    wid = sid * NC + cid

    pltpu.async_copy(sa_hbm, as_v, sem0)
    pltpu.async_copy(sd_hbm, ad_v, sem1)

    @plsc.parallel_loop(0, N // 16, step=1, unroll=8)
    def _(i):
        den_v[pl.ds(i * 16, 16)] = jnp.zeros((16,), jnp.float32)

    @plsc.parallel_loop(0, C, step=1, unroll=8)
    def _(r):
        for cb in range(D // 16):
            rows0_v[r, pl.ds(cb * 16, 16)] = jnp.zeros((16,), jnp.float32)

    pltpu.make_async_copy(sa_hbm, as_v, sem0).wait()
    pltpu.make_async_copy(sd_hbm, ad_v, sem1).wait()

    @pl.loop(0, RPS // C)
    def _(j):
        pltpu.sync_copy(rows0_v, acc.at[pl.ds(sid * RPS + j * C, C)])

    pltpu.sync_copy(rows0_v.at[pl.ds(0, RPS - (RPS // C) * C)],
                    acc.at[pl.ds(sid * RPS + (RPS // C) * C,
                                 RPS - (RPS // C) * C)])

    @pl.when(sid == NS - 1)
    def _():
        pltpu.sync_copy(rows0_v.at[pl.ds(0, N - NS * RPS)],
                        acc.at[pl.ds(NS * RPS, N - NS * RPS)])

    def _mbody(i, m):
        return jnp.maximum(m, as_v[pl.ds(i * 16, 16)])

    mvec = lax.fori_loop(0, N // 16, _mbody,
                         jnp.full((16,), -3e38, jnp.float32))
    amax = jnp.max(mvec)

    plsc.subcore_barrier()

    NCHUNK = EPW // C
    ibase = wid * NCHUNK

    def fetch(t, sidi, rows, sem):
        pltpu.sync_copy(eidx_hbm.at[ibase + t], sidi)
        pltpu.async_copy(h_hbm.at[sidi.at[0]], rows, sem)

    def process(sidi, rows, sem):
        for g in range(C // 16):
            sidx = sidi[0, pl.ds(g * 16, 16)]
            didx = sidi[1, pl.ds(g * 16, 16)]
            sv = plsc.load_gather(as_v, [sidx])
            dv = plsc.load_gather(ad_v, [didx])
            v = sv + dv
            e = jnp.maximum(v, 0.2 * v)
            w = amax + dv
            cmax = jnp.maximum(w, 0.2 * w)
            ex = jnp.exp(e - cmax)
            ex_v[pl.ds(g * 16, 16)] = ex
            plsc.addupdate_scatter(den_v, [didx], ex)

        pltpu.make_async_copy(h_hbm.at[sidi.at[0]], rows, sem).wait()

        @plsc.parallel_loop(0, C, step=1, unroll=8)
        def _(r):
            exb = plsc.load_gather(ex_v, [jnp.full((16,), r, jnp.int32)])
            for cb in range(D // 16):
                rows[r, pl.ds(cb * 16, 16)] = (
                    rows[r, pl.ds(cb * 16, 16)] * exb)

        pltpu.sync_copy(rows, acc.at[sidi.at[1]], add=True)

    fetch(0, sidi0_v, rows0_v, sem0)

    @pl.loop(0, (NCHUNK + 1) // 2)
    def _(p):
        t0 = 2 * p

        @pl.when(t0 + 1 <= NCHUNK - 1)
        def _():
            fetch(t0 + 1, sidi1_v, rows1_v, sem1)

        process(sidi0_v, rows0_v, sem0)

        @pl.when(t0 + 2 <= NCHUNK - 1)
        def _():
            fetch(t0 + 2, sidi0_v, rows0_v, sem0)

        @pl.when(t0 + 1 <= NCHUNK - 1)
        def _():
            process(sidi1_v, rows1_v, sem1)

    plsc.subcore_barrier()

    pltpu.async_copy(acc.at[pl.ds(sid * RPS, RPS)],
                     num_hbm.at[cid, pl.ds(sid * RPS, RPS)], sem0)
    pltpu.async_copy(den_v, den_hbm.at[cid, sid], sem1)

    @pl.when(sid == NS - 1)
    def _():
        pltpu.sync_copy(acc.at[pl.ds(NS * RPS, N - NS * RPS)],
                        num_hbm.at[cid, pl.ds(NS * RPS, N - NS * RPS)])

    pltpu.make_async_copy(acc.at[pl.ds(sid * RPS, RPS)],
                          num_hbm.at[cid, pl.ds(sid * RPS, RPS)], sem0).wait()
    pltpu.make_async_copy(den_v, den_hbm.at[cid, sid], sem1).wait()


def _sc_edge(h, eidx, sa, sd):
    mesh = plsc.VectorSubcoreMesh(core_axis_name="c", subcore_axis_name="s")
    k = functools.partial(
        pl.kernel,
        out_type=[
            jax.ShapeDtypeStruct((NC, N, D), jnp.float32),
            jax.ShapeDtypeStruct((NC, NS, N), jnp.float32),
        ],
        mesh=mesh,
        compiler_params=pltpu.CompilerParams(needs_layout_passes=False,
                                             use_tc_tiling_on_sc=False),
        scratch_types=[
            pltpu.VMEM((N,), jnp.float32),
            pltpu.VMEM((N,), jnp.float32),
            pltpu.VMEM((2, C), jnp.int32),
            pltpu.VMEM((2, C), jnp.int32),
            pltpu.VMEM((C,), jnp.float32),
            pltpu.VMEM((C, D), jnp.float32),
            pltpu.VMEM((C, D), jnp.float32),
            pltpu.VMEM((N,), jnp.float32),
            pltpu.VMEM_SHARED((N, D), jnp.float32),
            pltpu.SemaphoreType.DMA,
            pltpu.SemaphoreType.DMA,
        ],
    )(_sc_body)
    return k(h, eidx, sa, sd)


def _combine_body(p_ref, dens_ref, bias_ref, out_ref):
    num = p_ref[0] + p_ref[1]
    den = jnp.sum(dens_ref[...], axis=(0, 1))
    out_ref[...] = num / (den[:, None] + 1e-16) + bias_ref[...]


def _combine(parts, dens, bias2d):
    return pl.pallas_call(
        _combine_body,
        out_shape=jax.ShapeDtypeStruct((N, D), jnp.float32),
    )(parts, dens, bias2d)


def kernel(x, edge_index, W, a_src, a_dst, bias):
    eidx = jnp.transpose(edge_index.reshape(2, E // C, C), (1, 0, 2))
    h, sa, sd = _project(x, W, a_src.reshape(D, 1), a_dst.reshape(D, 1))
    parts, dens = _sc_edge(h, eidx, sa.reshape(N), sd.reshape(N))
    return _combine(parts, dens, bias.reshape(1, D))
